# baseline (speedup 1.0000x reference)
import functools
import numpy as np
import jax
import jax.numpy as jnp
from jax import lax
from jax.experimental import pallas as pl
from jax.experimental.pallas import tpu as pltpu
from jax.experimental.pallas import tpu_sc as plsc

_IN_D = 74
_OUT_D = 157
_ROWS = 1024 * 256
_NW = 32
_ROWS_PER_W = _ROWS // _NW
_CHUNK = 128
_N_CHUNK = _ROWS_PER_W // _CHUNK
_GROUPS = _CHUNK // 16

_BUCKETS = [
    (14, 20, True, 1500.0),
    (15, 20, True, 1500.0),
    (19, 16, True, 3000.0),
    (56, 16, False, 120.0),
    (57, 16, False, 120.0),
]
_BUCKET_COLS = frozenset(c for c, _, _, _ in _BUCKETS)
_PASSTHROUGH = [c for c in range(_IN_D) if c not in _BUCKET_COLS]


def _bucket_plan():
    pass_pairs = [(j, s) for j, s in enumerate(_PASSTHROUGH)]
    onehot = []
    out_c = len(_PASSTHROUGH)
    for src, nb, is_sqrt, mx in _BUCKETS:
        if is_sqrt:
            thr = [(t / (nb - 1)) ** 2 * mx for t in range(nb)]
        else:
            thr = [t / (nb - 1) * mx for t in range(nb)]
        lo = [-np.inf] + [np.float32(t) for t in thr[1:]]
        hi = [np.float32(t) for t in thr[1:]] + [np.inf]
        for t in range(nb):
            onehot.append((out_c, src, float(lo[t]), float(hi[t])))
            out_c += 1
    assert out_c == _OUT_D
    return pass_pairs, onehot


_PASS_PAIRS, _ONEHOT_COLS = _bucket_plan()


def _compute_chunk(in_ref, out_ref):
    iota = lax.iota(jnp.int32, 16)
    riota_in = iota * _IN_D
    riota_out = iota * _OUT_D

    def group_body(g, carry):
        rb_in = riota_in + g * (16 * _IN_D)
        rb_out = riota_out + g * (16 * _OUT_D)
        src_vecs = {}
        for src, _, _, _ in _BUCKETS:
            if src not in src_vecs:
                src_vecs[src] = plsc.load_gather(in_ref, [rb_in + src])
        for out_c, src_c in _PASS_PAIRS:
            v = plsc.load_gather(in_ref, [rb_in + src_c])
            plsc.store_scatter(out_ref, [rb_out + out_c], v)
        one = jnp.float32(1.0)
        zero = jnp.float32(0.0)
        for out_c, src_c, lo, hi in _ONEHOT_COLS:
            v = src_vecs[src_c]
            if np.isinf(lo):
                m = v < hi
            elif np.isinf(hi):
                m = v >= lo
            else:
                m = (v >= lo) & (v < hi)
            plsc.store_scatter(out_ref, [rb_out + out_c], jnp.where(m, one, zero))
        return carry

    lax.fori_loop(0, _GROUPS, group_body, 0)


def _sc_body(in_hbm, out_hbm, in_buf0, in_buf1, out_buf0, out_buf1,
             in_sem0, in_sem1, out_sem0, out_sem1):
    nc = 2
    wid = lax.axis_index("s") * nc + lax.axis_index("c")
    wbase = wid * _ROWS_PER_W
    in_bufs = [in_buf0, in_buf1]
    out_bufs = [out_buf0, out_buf1]
    in_sems = [in_sem0, in_sem1]
    out_sems = [out_sem0, out_sem1]

    def in_slice(ci):
        return in_hbm.at[pl.ds((wbase + ci * _CHUNK) * _IN_D, _CHUNK * _IN_D)]

    def out_slice(ci):
        return out_hbm.at[pl.ds((wbase + ci * _CHUNK) * _OUT_D, _CHUNK * _OUT_D)]

    pltpu.async_copy(in_slice(0), in_bufs[0], in_sems[0])
    pltpu.async_copy(in_slice(1), in_bufs[1], in_sems[1])

    def outer(i, carry):
        for b in range(2):
            ci = i * 2 + b
            pltpu.make_async_copy(in_slice(ci), in_bufs[b], in_sems[b]).wait()
            @pl.when(i >= 1)
            def _():
                pltpu.make_async_copy(out_bufs[b], out_slice(ci),
                                      out_sems[b]).wait()
            _compute_chunk(in_bufs[b], out_bufs[b])
            @pl.when(i < _N_CHUNK // 2 - 1)
            def _():
                pltpu.async_copy(in_slice(ci + 2), in_bufs[b], in_sems[b])
            pltpu.async_copy(out_bufs[b], out_slice(ci), out_sems[b])
        return carry

    lax.fori_loop(0, _N_CHUNK // 2, outer, 0)
    for b in range(2):
        ci = _N_CHUNK - 2 + b
        pltpu.make_async_copy(out_bufs[b], out_slice(ci), out_sems[b]).wait()


@jax.jit
def _preprocess(flat_in):
    mesh = plsc.VectorSubcoreMesh(core_axis_name="c", subcore_axis_name="s")
    k = pl.kernel(
        _sc_body,
        out_type=jax.ShapeDtypeStruct((_ROWS * _OUT_D,), jnp.float32),
        mesh=mesh,
        scratch_types=[
            pltpu.VMEM((_CHUNK * _IN_D,), jnp.float32),
            pltpu.VMEM((_CHUNK * _IN_D,), jnp.float32),
            pltpu.VMEM((_CHUNK * _OUT_D,), jnp.float32),
            pltpu.VMEM((_CHUNK * _OUT_D,), jnp.float32),
            pltpu.SemaphoreType.DMA,
            pltpu.SemaphoreType.DMA,
            pltpu.SemaphoreType.DMA,
            pltpu.SemaphoreType.DMA,
        ],
        compiler_params=pltpu.CompilerParams(needs_layout_passes=False),
    )
    return k(flat_in)


def kernel(features):
    flat = features.astype(jnp.float32).reshape(-1)
    out = _preprocess(flat)
    return out.reshape(features.shape[:-1] + (_OUT_D,))
